# Initial kernel scaffold; baseline (speedup 1.0000x reference)
import jax
import jax.numpy as jnp
from jax.experimental import pallas as pl


def kernel(x, edge_index, batch, W1, b1, W2, b2, W3, b3, A1, ab1, A2, ab2, C1, cb1, C2, cb2):
    raise NotImplementedError("write your pallas kernel here")


# speedup vs baseline: 6.3202x; 6.3202x over previous
import functools

import jax
import jax.numpy as jnp
from jax import lax
from jax.experimental import pallas as pl
from jax.experimental.pallas import tpu as pltpu
import jax.experimental.pallas.tpu_sc as plsc

N_NODES = 10000
E_EDGES = 320000
F_IN = 128
H = 128
A_DIM = 40
G = 256

NC = 2
NS = 16
NW = NC * NS
CHUNK = 128
NP = 10240
EP = 327680
KW = EP // (NW * CHUNK)
RPS = NP // NS
BR = 2560


def _deg_body(dstm_hbm, ones_hbm, zeros_hbm, out_hbm, didx, rows, acc, sem):
    c = lax.axis_index("c")
    s = lax.axis_index("s")
    wid = s * NC + c

    pltpu.sync_copy(zeros_hbm.at[pl.ds(s * RPS, RPS)],
                    acc.at[pl.ds(s * RPS, RPS)])
    pltpu.sync_copy(ones_hbm, rows)
    pltpu.sync_copy(dstm_hbm.at[pl.ds(wid * KW, KW)], didx)
    plsc.subcore_barrier()

    def step(j, carry):
        pltpu.sync_copy(rows, acc.at[didx.at[j]], add=True)
        return carry

    lax.fori_loop(0, KW, step, 0)
    plsc.subcore_barrier()
    pltpu.sync_copy(acc.at[pl.ds(s * RPS, RPS)],
                    out_hbm.at[c].at[pl.ds(s * RPS, RPS)])


@functools.cache
def _deg_kernel():
    return pl.kernel(
        _deg_body,
        out_type=jax.ShapeDtypeStruct((NC, NP, 128), jnp.float32),
        mesh=plsc.VectorSubcoreMesh(core_axis_name="c", subcore_axis_name="s",
                                    num_cores=NC, num_subcores=NS),
        scratch_types=[
            pltpu.VMEM((KW, CHUNK), jnp.int32),
            pltpu.VMEM((CHUNK, 128), jnp.float32),
            pltpu.VMEM_SHARED((NP, 128), jnp.float32),
            pltpu.SemaphoreType.DMA,
        ],
    )


def _deg_call(dstm, ones_c, zeros_n):
    return _deg_kernel()(dstm, ones_c, zeros_n)


def _scat_body(vs_hbm, srcm_hbm, dstm_hbm, zeros_hbm, out_hbm,
               sidx, didx, rows, acc, sem):
    c = lax.axis_index("c")
    s = lax.axis_index("s")
    wid = s * NC + c

    pltpu.sync_copy(zeros_hbm.at[pl.ds(s * RPS, RPS)],
                    acc.at[pl.ds(s * RPS, RPS)])
    base = wid * KW
    pltpu.sync_copy(srcm_hbm.at[pl.ds(base, KW)], sidx)
    pltpu.sync_copy(dstm_hbm.at[pl.ds(base, KW)], didx)
    plsc.subcore_barrier()

    def step(j, carry):
        pltpu.async_copy(vs_hbm.at[sidx.at[j]], rows, sem).wait()
        pltpu.sync_copy(rows, acc.at[didx.at[j]], add=True)
        return carry

    lax.fori_loop(0, KW, step, 0)
    plsc.subcore_barrier()
    pltpu.sync_copy(acc.at[pl.ds(s * RPS, RPS)],
                    out_hbm.at[c].at[pl.ds(s * RPS, RPS)])


@functools.cache
def _scat_kernel():
    return pl.kernel(
        _scat_body,
        out_type=jax.ShapeDtypeStruct((NC, NP, 128), jnp.float32),
        mesh=plsc.VectorSubcoreMesh(core_axis_name="c", subcore_axis_name="s",
                                    num_cores=NC, num_subcores=NS),
        scratch_types=[
            pltpu.VMEM((KW, CHUNK), jnp.int32),
            pltpu.VMEM((KW, CHUNK), jnp.int32),
            pltpu.VMEM((CHUNK, 128), jnp.float32),
            pltpu.VMEM_SHARED((NP, 128), jnp.float32),
            pltpu.SemaphoreType.DMA,
        ],
    )


def _scat_call(vs, srcm, dstm, zeros_nd):
    return _scat_kernel()(vs, srcm, dstm, zeros_nd)


def _dis_body(hists_ref, x_ref, dis_ref, vs0_ref):
    deg = hists_ref[0, :, 0:1] + hists_ref[1, :, 0:1] + 1.0
    dis = lax.rsqrt(deg)
    dis_ref[...] = dis
    vs0_ref[...] = x_ref[...] * dis


def _dis_call(hists, xp):
    return pl.pallas_call(
        _dis_body,
        out_shape=(jax.ShapeDtypeStruct((NP, 1), jnp.float32),
                   jax.ShapeDtypeStruct((NP, 128), jnp.float32)),
    )(hists, xp)


def _layer12_body(s_ref, vs_ref, dis_ref, w_ref, b_ref, outa_ref, outb_ref):
    dis = dis_ref[...]
    p = dis * (s_ref[0] + s_ref[1] + vs_ref[...])
    t = jnp.maximum(
        jnp.dot(p, w_ref[...], preferred_element_type=jnp.float32)
        + b_ref[...], 0.0)
    tsc = t * dis
    fo = w_ref.shape[1]
    outa_ref[...] = tsc[:, :fo // 2] if fo == 256 else tsc
    if fo == 256:
        outb_ref[...] = tsc[:, fo // 2:]


def _layer1_call(s, vs, dis, w, b):
    body = lambda s_ref, vs_ref, dis_ref, w_ref, b_ref, out_ref: \
        _layer12_body(s_ref, vs_ref, dis_ref, w_ref, b_ref, out_ref, None)
    return pl.pallas_call(
        body,
        grid=(NP // BR,),
        in_specs=[
            pl.BlockSpec((NC, BR, 128), lambda i: (0, i, 0)),
            pl.BlockSpec((BR, 128), lambda i: (i, 0)),
            pl.BlockSpec((BR, 1), lambda i: (i, 0)),
            pl.BlockSpec((128, 128), lambda i: (0, 0)),
            pl.BlockSpec((1, 128), lambda i: (0, 0)),
        ],
        out_specs=pl.BlockSpec((BR, 128), lambda i: (i, 0)),
        out_shape=jax.ShapeDtypeStruct((NP, 128), jnp.float32),
    )(s, vs, dis, w, b)


def _layer2_call(s, vs, dis, w, b):
    return pl.pallas_call(
        _layer12_body,
        grid=(NP // BR,),
        in_specs=[
            pl.BlockSpec((NC, BR, 128), lambda i: (0, i, 0)),
            pl.BlockSpec((BR, 128), lambda i: (i, 0)),
            pl.BlockSpec((BR, 1), lambda i: (i, 0)),
            pl.BlockSpec((128, 256), lambda i: (0, 0)),
            pl.BlockSpec((1, 256), lambda i: (0, 0)),
        ],
        out_specs=(pl.BlockSpec((BR, 128), lambda i: (i, 0)),
                   pl.BlockSpec((BR, 128), lambda i: (i, 0))),
        out_shape=(jax.ShapeDtypeStruct((NP, 128), jnp.float32),
                   jax.ShapeDtypeStruct((NP, 128), jnp.float32)),
    )(s, vs, dis, w, b)


def _layer3_body(sa_ref, sb_ref, vsa_ref, vsb_ref, dis_ref, w_ref, b_ref,
                 out_ref):
    dis = dis_ref[...]
    pa = dis * (sa_ref[0] + sa_ref[1] + vsa_ref[...])
    pb = dis * (sb_ref[0] + sb_ref[1] + vsb_ref[...])
    p = jnp.concatenate([pa, pb], axis=1)
    out_ref[...] = jnp.maximum(
        jnp.dot(p, w_ref[...], preferred_element_type=jnp.float32)
        + b_ref[...], 0.0)


def _layer3_call(sa, sb, vsa, vsb, dis, w, b):
    return pl.pallas_call(
        _layer3_body,
        grid=(NP // BR,),
        in_specs=[
            pl.BlockSpec((NC, BR, 128), lambda i: (0, i, 0)),
            pl.BlockSpec((NC, BR, 128), lambda i: (0, i, 0)),
            pl.BlockSpec((BR, 128), lambda i: (i, 0)),
            pl.BlockSpec((BR, 128), lambda i: (i, 0)),
            pl.BlockSpec((BR, 1), lambda i: (i, 0)),
            pl.BlockSpec((256, 256), lambda i: (0, 0)),
            pl.BlockSpec((1, 256), lambda i: (0, 0)),
        ],
        out_specs=pl.BlockSpec((BR, 256), lambda i: (i, 0)),
        out_shape=jax.ShapeDtypeStruct((NP, 256), jnp.float32),
    )(sa, sb, vsa, vsb, dis, w, b)


def _pool_body(t_ref, batch_ref, a1_ref, ab1_ref, a2_ref, ab2_ref,
               c1_ref, cb1_ref, c2_ref, cb2_ref,
               logits_ref, value_ref, sums, cnts):
    i = pl.program_id(0)

    @pl.when(i == 0)
    def _():
        sums[...] = jnp.zeros_like(sums)
        cnts[...] = jnp.zeros_like(cnts)

    onehot = (batch_ref[...] ==
              lax.broadcasted_iota(jnp.int32, (1, G), 1)).astype(jnp.float32)
    t = t_ref[...]
    sums[...] += lax.dot_general(onehot, t, (((0,), (0,)), ((), ())),
                                 preferred_element_type=jnp.float32)
    cnts[...] += lax.dot_general(onehot, jnp.ones((BR, 128), jnp.float32),
                                 (((0,), (0,)), ((), ())),
                                 preferred_element_type=jnp.float32)

    @pl.when(i == pl.num_programs(0) - 1)
    def _():
        g = sums[...] / jnp.maximum(cnts[:, 0:1], 1.0)
        ah = jnp.maximum(
            jnp.dot(g, a1_ref[...], preferred_element_type=jnp.float32)
            + ab1_ref[...], 0.0)
        logits_ref[...] = (
            jnp.dot(ah, a2_ref[...], preferred_element_type=jnp.float32)
            + ab2_ref[...])
        ch = jnp.maximum(
            jnp.dot(g, c1_ref[...], preferred_element_type=jnp.float32)
            + cb1_ref[...], 0.0)
        value_ref[...] = (
            jnp.dot(ch, c2_ref[...], preferred_element_type=jnp.float32)
            + cb2_ref[...])


def _pool_call(t3, batchp, a1, ab1, a2, ab2, c1, cb1, c2, cb2):
    full = lambda shape: pl.BlockSpec(shape, lambda i: tuple(0 for _ in shape))
    return pl.pallas_call(
        _pool_body,
        grid=(NP // BR,),
        in_specs=[
            pl.BlockSpec((BR, 256), lambda i: (i, 0)),
            pl.BlockSpec((BR, 1), lambda i: (i, 0)),
            full((256, 128)), full((1, 128)), full((128, A_DIM)),
            full((1, A_DIM)), full((256, 128)), full((1, 128)),
            full((128, 1)), full((1, 1)),
        ],
        out_specs=(full((G, A_DIM)), full((G, 1))),
        out_shape=(jax.ShapeDtypeStruct((G, A_DIM), jnp.float32),
                   jax.ShapeDtypeStruct((G, 1), jnp.float32)),
        scratch_shapes=[
            pltpu.VMEM((G, 256), jnp.float32),
            pltpu.VMEM((G, 128), jnp.float32),
        ],
    )(t3, batchp, a1, ab1, a2, ab2, c1, cb1, c2, cb2)


def kernel(x, edge_index, batch, W1, b1, W2, b2, W3, b3,
           A1, ab1, A2, ab2, C1, cb1, C2, cb2):
    src = edge_index[0]
    dst = edge_index[1]
    epad = EP - E_EDGES
    srcp = jnp.concatenate([src, jnp.zeros((epad,), jnp.int32)])
    dstp = jnp.concatenate([dst, jnp.full((epad,), N_NODES, jnp.int32)])
    srcm = srcp.reshape(EP // CHUNK, CHUNK)
    dstm = dstp.reshape(EP // CHUNK, CHUNK)
    xp = jnp.pad(x, ((0, NP - N_NODES), (0, 0)))
    batchp = jnp.concatenate(
        [batch, jnp.full((NP - N_NODES,), G, jnp.int32)]).reshape(NP, 1)
    zeros_nd = jnp.zeros((NP, 128), jnp.float32)

    hists = _deg_call(dstm, jnp.ones((CHUNK, 128), jnp.float32), zeros_nd)
    dis, vs0 = _dis_call(hists, xp)

    s1 = _scat_call(vs0, srcm, dstm, zeros_nd)
    vs1 = _layer1_call(s1, vs0, dis, W1, b1.reshape(1, H))

    s2 = _scat_call(vs1, srcm, dstm, zeros_nd)
    vs2a, vs2b = _layer2_call(s2, vs1, dis, W2, b2.reshape(1, 2 * H))

    s3a = _scat_call(vs2a, srcm, dstm, zeros_nd)
    s3b = _scat_call(vs2b, srcm, dstm, zeros_nd)
    t3 = _layer3_call(s3a, s3b, vs2a, vs2b, dis, W3, b3.reshape(1, 2 * H))

    logits, value = _pool_call(
        t3, batchp, A1, ab1.reshape(1, H), A2, ab2.reshape(1, A_DIM),
        C1, cb1.reshape(1, H), C2, cb2.reshape(1, 1))
    return (logits, value)


# speedup vs baseline: 7.0265x; 1.1118x over previous
import functools

import jax
import jax.numpy as jnp
from jax import lax
from jax.experimental import pallas as pl
from jax.experimental.pallas import tpu as pltpu
import jax.experimental.pallas.tpu_sc as plsc

N_NODES = 10000
E_EDGES = 320000
F_IN = 128
H = 128
A_DIM = 40
G = 256

NC = 2
NS = 16
NW = NC * NS
CHUNK = 128
NP = 10240
EP = 327680
KW = EP // (NW * CHUNK)
RPS = NP // NS
BR = 2560


def _deg_body(dstm_hbm, ones_hbm, zeros_hbm, out_hbm, didx, rows, acc, sem):
    c = lax.axis_index("c")
    s = lax.axis_index("s")
    wid = s * NC + c

    pltpu.sync_copy(zeros_hbm.at[pl.ds(s * RPS, RPS)],
                    acc.at[pl.ds(s * RPS, RPS)])
    pltpu.sync_copy(ones_hbm, rows)
    pltpu.sync_copy(dstm_hbm.at[pl.ds(wid * KW, KW)], didx)
    plsc.subcore_barrier()

    def step(j, carry):
        pltpu.sync_copy(rows, acc.at[didx.at[j]], add=True)
        return carry

    lax.fori_loop(0, KW, step, 0)
    plsc.subcore_barrier()
    pltpu.sync_copy(acc.at[pl.ds(s * RPS, RPS)],
                    out_hbm.at[c].at[pl.ds(s * RPS, RPS)])


@functools.cache
def _deg_kernel():
    return pl.kernel(
        _deg_body,
        out_type=jax.ShapeDtypeStruct((NC, NP, 128), jnp.float32),
        mesh=plsc.VectorSubcoreMesh(core_axis_name="c", subcore_axis_name="s",
                                    num_cores=NC, num_subcores=NS),
        scratch_types=[
            pltpu.VMEM((KW, CHUNK), jnp.int32),
            pltpu.VMEM((CHUNK, 128), jnp.float32),
            pltpu.VMEM_SHARED((NP, 128), jnp.float32),
            pltpu.SemaphoreType.DMA,
        ],
    )


def _deg_call(dstm, ones_c, zeros_n):
    return _deg_kernel()(dstm, ones_c, zeros_n)


NB = 2


KW2 = KW // 2


def _scat_body(vs_hbm, srcm_hbm, dstm_hbm, zeros_hbm, out_hbm,
               sidx, didx, rows, acc, sem0, sem1):
    c = lax.axis_index("c")
    s = lax.axis_index("s")
    wid = s * NC + c
    sems = (sem0, sem1)

    pltpu.sync_copy(zeros_hbm.at[pl.ds(s * RPS, RPS)],
                    acc.at[pl.ds(s * RPS, RPS)])
    base = wid * KW
    plsc.subcore_barrier()

    for h in range(2):
        pltpu.sync_copy(srcm_hbm.at[pl.ds(base + h * KW2, KW2)], sidx)
        pltpu.sync_copy(dstm_hbm.at[pl.ds(base + h * KW2, KW2)], didx)

        for b in range(NB):
            pltpu.async_copy(vs_hbm.at[sidx.at[b]], rows.at[b], sems[b])

        def step(g, carry):
            for b in range(NB):
                j = g * NB + b
                pltpu.make_async_copy(vs_hbm.at[sidx.at[0]], rows.at[b],
                                      sems[b]).wait()
                pltpu.sync_copy(rows.at[b], acc.at[didx.at[j]], add=True)

                @pl.when(j + NB < KW2)
                def _():
                    pltpu.async_copy(vs_hbm.at[sidx.at[j + NB]], rows.at[b],
                                     sems[b])
            return carry

        lax.fori_loop(0, KW2 // NB, step, 0)

    plsc.subcore_barrier()
    pltpu.sync_copy(acc.at[pl.ds(s * RPS, RPS)],
                    out_hbm.at[c].at[pl.ds(s * RPS, RPS)])


@functools.cache
def _scat_kernel():
    return pl.kernel(
        _scat_body,
        out_type=jax.ShapeDtypeStruct((NC, NP, 128), jnp.float32),
        mesh=plsc.VectorSubcoreMesh(core_axis_name="c", subcore_axis_name="s",
                                    num_cores=NC, num_subcores=NS),
        scratch_types=[
            pltpu.VMEM((KW2, CHUNK), jnp.int32),
            pltpu.VMEM((KW2, CHUNK), jnp.int32),
            pltpu.VMEM((NB, CHUNK, 128), jnp.float32),
            pltpu.VMEM_SHARED((NP, 128), jnp.float32),
            pltpu.SemaphoreType.DMA,
            pltpu.SemaphoreType.DMA,
        ],
    )


def _scat_call(vs, srcm, dstm, zeros_nd):
    return _scat_kernel()(vs, srcm, dstm, zeros_nd)


def _dis_body(hists_ref, x_ref, dis_ref, vs0_ref):
    deg = hists_ref[0, :, 0:1] + hists_ref[1, :, 0:1] + 1.0
    dis = lax.rsqrt(deg)
    dis_ref[...] = dis
    vs0_ref[...] = x_ref[...] * dis


def _dis_call(hists, xp):
    return pl.pallas_call(
        _dis_body,
        out_shape=(jax.ShapeDtypeStruct((NP, 1), jnp.float32),
                   jax.ShapeDtypeStruct((NP, 128), jnp.float32)),
    )(hists, xp)


def _layer12_body(s_ref, vs_ref, dis_ref, w_ref, b_ref, outa_ref, outb_ref):
    dis = dis_ref[...]
    p = dis * (s_ref[0] + s_ref[1] + vs_ref[...])
    t = jnp.maximum(
        jnp.dot(p, w_ref[...], preferred_element_type=jnp.float32)
        + b_ref[...], 0.0)
    tsc = t * dis
    fo = w_ref.shape[1]
    outa_ref[...] = tsc[:, :fo // 2] if fo == 256 else tsc
    if fo == 256:
        outb_ref[...] = tsc[:, fo // 2:]


def _layer1_call(s, vs, dis, w, b):
    body = lambda s_ref, vs_ref, dis_ref, w_ref, b_ref, out_ref: \
        _layer12_body(s_ref, vs_ref, dis_ref, w_ref, b_ref, out_ref, None)
    return pl.pallas_call(
        body,
        grid=(NP // BR,),
        in_specs=[
            pl.BlockSpec((NC, BR, 128), lambda i: (0, i, 0)),
            pl.BlockSpec((BR, 128), lambda i: (i, 0)),
            pl.BlockSpec((BR, 1), lambda i: (i, 0)),
            pl.BlockSpec((128, 128), lambda i: (0, 0)),
            pl.BlockSpec((1, 128), lambda i: (0, 0)),
        ],
        out_specs=pl.BlockSpec((BR, 128), lambda i: (i, 0)),
        out_shape=jax.ShapeDtypeStruct((NP, 128), jnp.float32),
    )(s, vs, dis, w, b)


def _layer2_call(s, vs, dis, w, b):
    return pl.pallas_call(
        _layer12_body,
        grid=(NP // BR,),
        in_specs=[
            pl.BlockSpec((NC, BR, 128), lambda i: (0, i, 0)),
            pl.BlockSpec((BR, 128), lambda i: (i, 0)),
            pl.BlockSpec((BR, 1), lambda i: (i, 0)),
            pl.BlockSpec((128, 256), lambda i: (0, 0)),
            pl.BlockSpec((1, 256), lambda i: (0, 0)),
        ],
        out_specs=(pl.BlockSpec((BR, 128), lambda i: (i, 0)),
                   pl.BlockSpec((BR, 128), lambda i: (i, 0))),
        out_shape=(jax.ShapeDtypeStruct((NP, 128), jnp.float32),
                   jax.ShapeDtypeStruct((NP, 128), jnp.float32)),
    )(s, vs, dis, w, b)


def _layer3_body(sa_ref, sb_ref, vsa_ref, vsb_ref, dis_ref, w_ref, b_ref,
                 out_ref):
    dis = dis_ref[...]
    pa = dis * (sa_ref[0] + sa_ref[1] + vsa_ref[...])
    pb = dis * (sb_ref[0] + sb_ref[1] + vsb_ref[...])
    p = jnp.concatenate([pa, pb], axis=1)
    out_ref[...] = jnp.maximum(
        jnp.dot(p, w_ref[...], preferred_element_type=jnp.float32)
        + b_ref[...], 0.0)


def _layer3_call(sa, sb, vsa, vsb, dis, w, b):
    return pl.pallas_call(
        _layer3_body,
        grid=(NP // BR,),
        in_specs=[
            pl.BlockSpec((NC, BR, 128), lambda i: (0, i, 0)),
            pl.BlockSpec((NC, BR, 128), lambda i: (0, i, 0)),
            pl.BlockSpec((BR, 128), lambda i: (i, 0)),
            pl.BlockSpec((BR, 128), lambda i: (i, 0)),
            pl.BlockSpec((BR, 1), lambda i: (i, 0)),
            pl.BlockSpec((256, 256), lambda i: (0, 0)),
            pl.BlockSpec((1, 256), lambda i: (0, 0)),
        ],
        out_specs=pl.BlockSpec((BR, 256), lambda i: (i, 0)),
        out_shape=jax.ShapeDtypeStruct((NP, 256), jnp.float32),
    )(sa, sb, vsa, vsb, dis, w, b)


def _pool_body(t_ref, batch_ref, a1_ref, ab1_ref, a2_ref, ab2_ref,
               c1_ref, cb1_ref, c2_ref, cb2_ref,
               logits_ref, value_ref, sums, cnts):
    i = pl.program_id(0)

    @pl.when(i == 0)
    def _():
        sums[...] = jnp.zeros_like(sums)
        cnts[...] = jnp.zeros_like(cnts)

    onehot = (batch_ref[...] ==
              lax.broadcasted_iota(jnp.int32, (1, G), 1)).astype(jnp.float32)
    t = t_ref[...]
    sums[...] += lax.dot_general(onehot, t, (((0,), (0,)), ((), ())),
                                 preferred_element_type=jnp.float32)
    cnts[...] += lax.dot_general(onehot, jnp.ones((BR, 128), jnp.float32),
                                 (((0,), (0,)), ((), ())),
                                 preferred_element_type=jnp.float32)

    @pl.when(i == pl.num_programs(0) - 1)
    def _():
        g = sums[...] / jnp.maximum(cnts[:, 0:1], 1.0)
        ah = jnp.maximum(
            jnp.dot(g, a1_ref[...], preferred_element_type=jnp.float32)
            + ab1_ref[...], 0.0)
        logits_ref[...] = (
            jnp.dot(ah, a2_ref[...], preferred_element_type=jnp.float32)
            + ab2_ref[...])
        ch = jnp.maximum(
            jnp.dot(g, c1_ref[...], preferred_element_type=jnp.float32)
            + cb1_ref[...], 0.0)
        value_ref[...] = (
            jnp.dot(ch, c2_ref[...], preferred_element_type=jnp.float32)
            + cb2_ref[...])


def _pool_call(t3, batchp, a1, ab1, a2, ab2, c1, cb1, c2, cb2):
    full = lambda shape: pl.BlockSpec(shape, lambda i: tuple(0 for _ in shape))
    return pl.pallas_call(
        _pool_body,
        grid=(NP // BR,),
        in_specs=[
            pl.BlockSpec((BR, 256), lambda i: (i, 0)),
            pl.BlockSpec((BR, 1), lambda i: (i, 0)),
            full((256, 128)), full((1, 128)), full((128, A_DIM)),
            full((1, A_DIM)), full((256, 128)), full((1, 128)),
            full((128, 1)), full((1, 1)),
        ],
        out_specs=(full((G, A_DIM)), full((G, 1))),
        out_shape=(jax.ShapeDtypeStruct((G, A_DIM), jnp.float32),
                   jax.ShapeDtypeStruct((G, 1), jnp.float32)),
        scratch_shapes=[
            pltpu.VMEM((G, 256), jnp.float32),
            pltpu.VMEM((G, 128), jnp.float32),
        ],
    )(t3, batchp, a1, ab1, a2, ab2, c1, cb1, c2, cb2)


def kernel(x, edge_index, batch, W1, b1, W2, b2, W3, b3,
           A1, ab1, A2, ab2, C1, cb1, C2, cb2):
    src = edge_index[0]
    dst = edge_index[1]
    epad = EP - E_EDGES
    srcp = jnp.concatenate([src, jnp.zeros((epad,), jnp.int32)])
    dstp = jnp.concatenate([dst, jnp.full((epad,), N_NODES, jnp.int32)])
    srcm = srcp.reshape(EP // CHUNK, CHUNK)
    dstm = dstp.reshape(EP // CHUNK, CHUNK)
    xp = jnp.pad(x, ((0, NP - N_NODES), (0, 0)))
    batchp = jnp.concatenate(
        [batch, jnp.full((NP - N_NODES,), G, jnp.int32)]).reshape(NP, 1)
    zeros_nd = jnp.zeros((NP, 128), jnp.float32)

    hists = _deg_call(dstm, jnp.ones((CHUNK, 128), jnp.float32), zeros_nd)
    dis, vs0 = _dis_call(hists, xp)

    s1 = _scat_call(vs0, srcm, dstm, zeros_nd)
    vs1 = _layer1_call(s1, vs0, dis, W1, b1.reshape(1, H))

    s2 = _scat_call(vs1, srcm, dstm, zeros_nd)
    vs2a, vs2b = _layer2_call(s2, vs1, dis, W2, b2.reshape(1, 2 * H))

    s3a = _scat_call(vs2a, srcm, dstm, zeros_nd)
    s3b = _scat_call(vs2b, srcm, dstm, zeros_nd)
    t3 = _layer3_call(s3a, s3b, vs2a, vs2b, dis, W3, b3.reshape(1, 2 * H))

    logits, value = _pool_call(
        t3, batchp, A1, ab1.reshape(1, H), A2, ab2.reshape(1, A_DIM),
        C1, cb1.reshape(1, H), C2, cb2.reshape(1, 1))
    return (logits, value)


# speedup vs baseline: 7.4567x; 1.0612x over previous
import functools

import jax
import jax.numpy as jnp
from jax import lax
from jax.experimental import pallas as pl
from jax.experimental.pallas import tpu as pltpu
import jax.experimental.pallas.tpu_sc as plsc

N_NODES = 10000
E_EDGES = 320000
F_IN = 128
H = 128
A_DIM = 40
G = 256

NC = 2
NS = 16
NW = NC * NS
CHUNK = 128
NP = 10240
EP = 327680
KW = EP // (NW * CHUNK)
RPS = NP // NS
BR = 2560


def _deg_body(dstm_hbm, ones_hbm, zeros_hbm, out_hbm, didx, rows, acc, sem):
    c = lax.axis_index("c")
    s = lax.axis_index("s")
    wid = s * NC + c

    pltpu.sync_copy(zeros_hbm.at[pl.ds(s * RPS, RPS)],
                    acc.at[pl.ds(s * RPS, RPS)])
    pltpu.sync_copy(ones_hbm, rows)
    pltpu.sync_copy(dstm_hbm.at[pl.ds(wid * KW, KW)], didx)
    plsc.subcore_barrier()

    def step(j, carry):
        pltpu.sync_copy(rows, acc.at[didx.at[j]], add=True)
        return carry

    lax.fori_loop(0, KW, step, 0)
    plsc.subcore_barrier()
    pltpu.sync_copy(acc.at[pl.ds(s * RPS, RPS)],
                    out_hbm.at[c].at[pl.ds(s * RPS, RPS)])


@functools.cache
def _deg_kernel():
    return pl.kernel(
        _deg_body,
        out_type=jax.ShapeDtypeStruct((NC, NP, 128), jnp.float32),
        mesh=plsc.VectorSubcoreMesh(core_axis_name="c", subcore_axis_name="s",
                                    num_cores=NC, num_subcores=NS),
        scratch_types=[
            pltpu.VMEM((KW, CHUNK), jnp.int32),
            pltpu.VMEM((CHUNK, 128), jnp.float32),
            pltpu.VMEM_SHARED((NP, 128), jnp.float32),
            pltpu.SemaphoreType.DMA,
        ],
    )


def _deg_call(dstm, ones_c, zeros_n):
    return _deg_kernel()(dstm, ones_c, zeros_n)


NB = 2


FAST_CORE = 0
KA = 128
KB = 32
PH = 32
NPH_A = KA // PH
FAST_TOT = NS * KA
assert NS * (KA + KB) == EP // CHUNK


def _scat_body(vs_hbm, srcm_hbm, dstm_hbm, zeros_hbm, out_hbm,
               sidx, didx, rows, acc, sem0, sem1):
    c = lax.axis_index("c")
    s = lax.axis_index("s")
    sems = (sem0, sem1)

    pltpu.sync_copy(zeros_hbm.at[pl.ds(s * RPS, RPS)],
                    acc.at[pl.ds(s * RPS, RPS)])
    is_fast = c == FAST_CORE
    nph = jnp.where(is_fast, NPH_A, KB // PH)
    base = pl.multiple_of(
        jnp.where(is_fast, s * KA, FAST_TOT + s * KB), PH)
    plsc.subcore_barrier()

    for h in range(NPH_A):
        @pl.when(h < nph)
        def _():
            pltpu.sync_copy(srcm_hbm.at[pl.ds(base + h * PH, PH)], sidx)
            pltpu.sync_copy(dstm_hbm.at[pl.ds(base + h * PH, PH)], didx)

            for b in range(NB):
                pltpu.async_copy(vs_hbm.at[sidx.at[b]], rows.at[b], sems[b])

            def step(g, carry):
                for b in range(NB):
                    j = g * NB + b
                    pltpu.make_async_copy(vs_hbm.at[sidx.at[0]], rows.at[b],
                                          sems[b]).wait()
                    pltpu.sync_copy(rows.at[b], acc.at[didx.at[j]], add=True)

                    @pl.when(j + NB < PH)
                    def _():
                        pltpu.async_copy(vs_hbm.at[sidx.at[j + NB]],
                                         rows.at[b], sems[b])
                return carry

            lax.fori_loop(0, PH // NB, step, 0)

    plsc.subcore_barrier()
    pltpu.sync_copy(acc.at[pl.ds(s * RPS, RPS)],
                    out_hbm.at[c].at[pl.ds(s * RPS, RPS)])


@functools.cache
def _scat_kernel():
    return pl.kernel(
        _scat_body,
        out_type=jax.ShapeDtypeStruct((NC, NP, 128), jnp.float32),
        mesh=plsc.VectorSubcoreMesh(core_axis_name="c", subcore_axis_name="s",
                                    num_cores=NC, num_subcores=NS),
        scratch_types=[
            pltpu.VMEM((PH, CHUNK), jnp.int32),
            pltpu.VMEM((PH, CHUNK), jnp.int32),
            pltpu.VMEM((NB, CHUNK, 128), jnp.float32),
            pltpu.VMEM_SHARED((NP, 128), jnp.float32),
            pltpu.SemaphoreType.DMA,
            pltpu.SemaphoreType.DMA,
        ],
    )


def _scat_call(vs, srcm, dstm, zeros_nd):
    return _scat_kernel()(vs, srcm, dstm, zeros_nd)


def _dis_body(hists_ref, x_ref, dis_ref, vs0_ref):
    deg = hists_ref[0, :, 0:1] + hists_ref[1, :, 0:1] + 1.0
    dis = lax.rsqrt(deg)
    dis_ref[...] = dis
    vs0_ref[...] = x_ref[...] * dis


def _dis_call(hists, xp):
    return pl.pallas_call(
        _dis_body,
        out_shape=(jax.ShapeDtypeStruct((NP, 1), jnp.float32),
                   jax.ShapeDtypeStruct((NP, 128), jnp.float32)),
    )(hists, xp)


def _layer12_body(s_ref, vs_ref, dis_ref, w_ref, b_ref, outa_ref, outb_ref):
    dis = dis_ref[...]
    p = dis * (s_ref[0] + s_ref[1] + vs_ref[...])
    t = jnp.maximum(
        jnp.dot(p, w_ref[...], preferred_element_type=jnp.float32)
        + b_ref[...], 0.0)
    tsc = t * dis
    fo = w_ref.shape[1]
    outa_ref[...] = tsc[:, :fo // 2] if fo == 256 else tsc
    if fo == 256:
        outb_ref[...] = tsc[:, fo // 2:]


def _layer1_call(s, vs, dis, w, b):
    body = lambda s_ref, vs_ref, dis_ref, w_ref, b_ref, out_ref: \
        _layer12_body(s_ref, vs_ref, dis_ref, w_ref, b_ref, out_ref, None)
    return pl.pallas_call(
        body,
        grid=(NP // BR,),
        in_specs=[
            pl.BlockSpec((NC, BR, 128), lambda i: (0, i, 0)),
            pl.BlockSpec((BR, 128), lambda i: (i, 0)),
            pl.BlockSpec((BR, 1), lambda i: (i, 0)),
            pl.BlockSpec((128, 128), lambda i: (0, 0)),
            pl.BlockSpec((1, 128), lambda i: (0, 0)),
        ],
        out_specs=pl.BlockSpec((BR, 128), lambda i: (i, 0)),
        out_shape=jax.ShapeDtypeStruct((NP, 128), jnp.float32),
    )(s, vs, dis, w, b)


def _layer2_call(s, vs, dis, w, b):
    return pl.pallas_call(
        _layer12_body,
        grid=(NP // BR,),
        in_specs=[
            pl.BlockSpec((NC, BR, 128), lambda i: (0, i, 0)),
            pl.BlockSpec((BR, 128), lambda i: (i, 0)),
            pl.BlockSpec((BR, 1), lambda i: (i, 0)),
            pl.BlockSpec((128, 256), lambda i: (0, 0)),
            pl.BlockSpec((1, 256), lambda i: (0, 0)),
        ],
        out_specs=(pl.BlockSpec((BR, 128), lambda i: (i, 0)),
                   pl.BlockSpec((BR, 128), lambda i: (i, 0))),
        out_shape=(jax.ShapeDtypeStruct((NP, 128), jnp.float32),
                   jax.ShapeDtypeStruct((NP, 128), jnp.float32)),
    )(s, vs, dis, w, b)


def _layer3_body(sa_ref, sb_ref, vsa_ref, vsb_ref, dis_ref, w_ref, b_ref,
                 out_ref):
    dis = dis_ref[...]
    pa = dis * (sa_ref[0] + sa_ref[1] + vsa_ref[...])
    pb = dis * (sb_ref[0] + sb_ref[1] + vsb_ref[...])
    p = jnp.concatenate([pa, pb], axis=1)
    out_ref[...] = jnp.maximum(
        jnp.dot(p, w_ref[...], preferred_element_type=jnp.float32)
        + b_ref[...], 0.0)


def _layer3_call(sa, sb, vsa, vsb, dis, w, b):
    return pl.pallas_call(
        _layer3_body,
        grid=(NP // BR,),
        in_specs=[
            pl.BlockSpec((NC, BR, 128), lambda i: (0, i, 0)),
            pl.BlockSpec((NC, BR, 128), lambda i: (0, i, 0)),
            pl.BlockSpec((BR, 128), lambda i: (i, 0)),
            pl.BlockSpec((BR, 128), lambda i: (i, 0)),
            pl.BlockSpec((BR, 1), lambda i: (i, 0)),
            pl.BlockSpec((256, 256), lambda i: (0, 0)),
            pl.BlockSpec((1, 256), lambda i: (0, 0)),
        ],
        out_specs=pl.BlockSpec((BR, 256), lambda i: (i, 0)),
        out_shape=jax.ShapeDtypeStruct((NP, 256), jnp.float32),
    )(sa, sb, vsa, vsb, dis, w, b)


def _pool_body(t_ref, batch_ref, a1_ref, ab1_ref, a2_ref, ab2_ref,
               c1_ref, cb1_ref, c2_ref, cb2_ref,
               logits_ref, value_ref, sums, cnts):
    i = pl.program_id(0)

    @pl.when(i == 0)
    def _():
        sums[...] = jnp.zeros_like(sums)
        cnts[...] = jnp.zeros_like(cnts)

    onehot = (batch_ref[...] ==
              lax.broadcasted_iota(jnp.int32, (1, G), 1)).astype(jnp.float32)
    t = t_ref[...]
    sums[...] += lax.dot_general(onehot, t, (((0,), (0,)), ((), ())),
                                 preferred_element_type=jnp.float32)
    cnts[...] += lax.dot_general(onehot, jnp.ones((BR, 128), jnp.float32),
                                 (((0,), (0,)), ((), ())),
                                 preferred_element_type=jnp.float32)

    @pl.when(i == pl.num_programs(0) - 1)
    def _():
        g = sums[...] / jnp.maximum(cnts[:, 0:1], 1.0)
        ah = jnp.maximum(
            jnp.dot(g, a1_ref[...], preferred_element_type=jnp.float32)
            + ab1_ref[...], 0.0)
        logits_ref[...] = (
            jnp.dot(ah, a2_ref[...], preferred_element_type=jnp.float32)
            + ab2_ref[...])
        ch = jnp.maximum(
            jnp.dot(g, c1_ref[...], preferred_element_type=jnp.float32)
            + cb1_ref[...], 0.0)
        value_ref[...] = (
            jnp.dot(ch, c2_ref[...], preferred_element_type=jnp.float32)
            + cb2_ref[...])


def _pool_call(t3, batchp, a1, ab1, a2, ab2, c1, cb1, c2, cb2):
    full = lambda shape: pl.BlockSpec(shape, lambda i: tuple(0 for _ in shape))
    return pl.pallas_call(
        _pool_body,
        grid=(NP // BR,),
        in_specs=[
            pl.BlockSpec((BR, 256), lambda i: (i, 0)),
            pl.BlockSpec((BR, 1), lambda i: (i, 0)),
            full((256, 128)), full((1, 128)), full((128, A_DIM)),
            full((1, A_DIM)), full((256, 128)), full((1, 128)),
            full((128, 1)), full((1, 1)),
        ],
        out_specs=(full((G, A_DIM)), full((G, 1))),
        out_shape=(jax.ShapeDtypeStruct((G, A_DIM), jnp.float32),
                   jax.ShapeDtypeStruct((G, 1), jnp.float32)),
        scratch_shapes=[
            pltpu.VMEM((G, 256), jnp.float32),
            pltpu.VMEM((G, 128), jnp.float32),
        ],
    )(t3, batchp, a1, ab1, a2, ab2, c1, cb1, c2, cb2)


def kernel(x, edge_index, batch, W1, b1, W2, b2, W3, b3,
           A1, ab1, A2, ab2, C1, cb1, C2, cb2):
    src = edge_index[0]
    dst = edge_index[1]
    epad = EP - E_EDGES
    srcp = jnp.concatenate([src, jnp.zeros((epad,), jnp.int32)])
    dstp = jnp.concatenate([dst, jnp.full((epad,), N_NODES, jnp.int32)])
    srcm = srcp.reshape(EP // CHUNK, CHUNK)
    dstm = dstp.reshape(EP // CHUNK, CHUNK)
    xp = jnp.pad(x, ((0, NP - N_NODES), (0, 0)))
    batchp = jnp.concatenate(
        [batch, jnp.full((NP - N_NODES,), G, jnp.int32)]).reshape(NP, 1)
    zeros_nd = jnp.zeros((NP, 128), jnp.float32)

    hists = _deg_call(dstm, jnp.ones((CHUNK, 128), jnp.float32), zeros_nd)
    dis, vs0 = _dis_call(hists, xp)

    s1 = _scat_call(vs0, srcm, dstm, zeros_nd)
    vs1 = _layer1_call(s1, vs0, dis, W1, b1.reshape(1, H))

    s2 = _scat_call(vs1, srcm, dstm, zeros_nd)
    vs2a, vs2b = _layer2_call(s2, vs1, dis, W2, b2.reshape(1, 2 * H))

    s3a = _scat_call(vs2a, srcm, dstm, zeros_nd)
    s3b = _scat_call(vs2b, srcm, dstm, zeros_nd)
    t3 = _layer3_call(s3a, s3b, vs2a, vs2b, dis, W3, b3.reshape(1, 2 * H))

    logits, value = _pool_call(
        t3, batchp, A1, ab1.reshape(1, H), A2, ab2.reshape(1, A_DIM),
        C1, cb1.reshape(1, H), C2, cb2.reshape(1, 1))
    return (logits, value)
